# Initial kernel scaffold; baseline (speedup 1.0000x reference)
import jax
import jax.numpy as jnp
from jax.experimental import pallas as pl


def kernel(x, edge_index, W1, b1, W2, b2, W3, b3, W4, b4, W5, b5):
    raise NotImplementedError("write your pallas kernel here")


# speedup vs baseline: 14.3534x; 14.3534x over previous
import functools

import jax
import jax.numpy as jnp
from jax import lax
from jax.experimental import pallas as pl
from jax.experimental.pallas import tpu as pltpu
from jax.experimental.pallas import tpu_sc as plsc

N = 10000
NP = 10240
E = 320000
NC = 2
NS = 16
CHUNK = 128
CPT = 79
EPAD = NC * NS * CPT * CHUNK
ROWS_PT = NP // NS
DEGW = 16


def _sc_mesh():
    return plsc.VectorSubcoreMesh(core_axis_name="c", subcore_axis_name="s",
                                  num_cores=NC, num_subcores=NS)


def _zero_vmem_rows(ref, nrows, width):
    z = jnp.zeros((16,), jnp.float32)

    @pl.loop(0, nrows)
    def _(r):
        for cc in range(width // 16):
            ref[r, pl.ds(cc * 16, 16)] = z


@functools.partial(jax.jit, static_argnums=())
def _deg_partials(dst2d):
    @functools.partial(
        pl.kernel,
        mesh=_sc_mesh(),
        out_type=jax.ShapeDtypeStruct((NC, NP, DEGW), jnp.float32),
        scratch_types=[
            pltpu.VMEM((CPT, CHUNK), jnp.int32),
            pltpu.VMEM((CHUNK, DEGW), jnp.float32),
            pltpu.VMEM((CHUNK, DEGW), jnp.float32),
            pltpu.VMEM_SHARED((NP, DEGW), jnp.float32),
        ],
    )
    def k(dst_hbm, out_hbm, dst_v, ones_v, zero_v, acc_sh):
        c = lax.axis_index("c")
        s = lax.axis_index("s")
        w = c * NS + s
        one = jnp.ones((16,), jnp.float32)
        zer = jnp.zeros((16,), jnp.float32)

        @pl.loop(0, CHUNK)
        def _(r):
            ones_v[r, pl.ds(0, 16)] = one
            zero_v[r, pl.ds(0, 16)] = zer

        row0 = s * ROWS_PT
        for kk in range(ROWS_PT // CHUNK):
            pltpu.sync_copy(zero_v, acc_sh.at[pl.ds(row0 + kk * CHUNK, CHUNK)])
        pltpu.sync_copy(dst_hbm.at[w], dst_v)
        plsc.subcore_barrier()

        @pl.loop(0, CPT)
        def _(j):
            pltpu.sync_copy(ones_v, acc_sh.at[dst_v.at[j]], add=True)

        plsc.subcore_barrier()
        pltpu.sync_copy(acc_sh.at[pl.ds(row0, ROWS_PT)],
                        out_hbm.at[c, pl.ds(row0, ROWS_PT)])

    return k(dst2d)


def _scatter_call(D):
    @functools.partial(
        pl.kernel,
        mesh=_sc_mesh(),
        out_type=jax.ShapeDtypeStruct((NC, NP, D), jnp.float32),
        scratch_types=[
            pltpu.VMEM((CPT, CHUNK), jnp.int32),
            pltpu.VMEM((CPT, CHUNK), jnp.int32),
            pltpu.VMEM((CHUNK, D), jnp.float32),
            pltpu.VMEM_SHARED((NP, D), jnp.float32),
            pltpu.SemaphoreType.DMA,
        ],
    )
    def k(hd_hbm, src_hbm, dst_hbm, out_hbm, src_v, dst_v, rows_v, acc_sh, sem):
        c = lax.axis_index("c")
        s = lax.axis_index("s")
        w = c * NS + s
        _zero_vmem_rows(rows_v, CHUNK, D)
        row0 = s * ROWS_PT
        for kk in range(ROWS_PT // CHUNK):
            pltpu.sync_copy(rows_v, acc_sh.at[pl.ds(row0 + kk * CHUNK, CHUNK)])
        pltpu.sync_copy(src_hbm.at[w], src_v)
        pltpu.sync_copy(dst_hbm.at[w], dst_v)
        plsc.subcore_barrier()

        @pl.loop(0, CPT)
        def _(j):
            pltpu.async_copy(hd_hbm.at[src_v.at[j]], rows_v, sem).wait()
            pltpu.sync_copy(rows_v, acc_sh.at[dst_v.at[j]], add=True)

        plsc.subcore_barrier()
        pltpu.sync_copy(acc_sh.at[pl.ds(row0, ROWS_PT)],
                        out_hbm.at[c, pl.ds(row0, ROWS_PT)])

    return k


_scatter_cache = {}


def _scatter(hd, src2d, dst2d):
    D = hd.shape[1]
    if D not in _scatter_cache:
        _scatter_cache[D] = _scatter_call(D)
    return _scatter_cache[D](hd, src2d, dst2d)


_R = 2048


def _dinv_from_deg(degp):
    def body(d_ref, o_ref):
        i = pl.program_id(0)
        total = d_ref[0, :, 0:1] + d_ref[1, :, 0:1] + 1.0
        rowid = i * _R + lax.broadcasted_iota(jnp.int32, (_R, 1), 0)
        o_ref[...] = jnp.where(rowid < N, lax.rsqrt(total), 0.0)

    return pl.pallas_call(
        body,
        grid=(NP // _R,),
        in_specs=[pl.BlockSpec((NC, _R, DEGW), lambda i: (0, i, 0))],
        out_specs=pl.BlockSpec((_R, 1), lambda i: (i, 0)),
        out_shape=jax.ShapeDtypeStruct((NP, 1), jnp.float32),
    )(degp)


def _mm_scale(xp, W, dinv):
    K, D = W.shape

    def body(x_ref, w_ref, v_ref, o_ref):
        h = jnp.dot(x_ref[...], w_ref[...], preferred_element_type=jnp.float32)
        o_ref[...] = h * v_ref[...]

    return pl.pallas_call(
        body,
        grid=(NP // _R,),
        in_specs=[
            pl.BlockSpec((_R, K), lambda i: (i, 0)),
            pl.BlockSpec((K, D), lambda i: (0, 0)),
            pl.BlockSpec((_R, 1), lambda i: (i, 0)),
        ],
        out_specs=pl.BlockSpec((_R, D), lambda i: (i, 0)),
        out_shape=jax.ShapeDtypeStruct((NP, D), jnp.float32),
    )(xp, W, dinv)


def _combine_mm(s, hd, dinv, b, Ws):
    D = hd.shape[1]
    nw = len(Ws)

    def body(s0_ref, s1_ref, hd_ref, v_ref, b_ref, *rest):
        w_refs, o_refs = rest[:nw], rest[nw:]
        v = v_ref[...]
        t = v * (s0_ref[0] + s1_ref[0] + hd_ref[...]) + b_ref[...]
        t = jnp.maximum(t, 0.0)
        for w_ref, o_ref in zip(w_refs, o_refs):
            o_ref[...] = jnp.dot(t, w_ref[...],
                                 preferred_element_type=jnp.float32) * v

    in_specs = [
        pl.BlockSpec((1, _R, D), lambda i: (0, i, 0)),
        pl.BlockSpec((1, _R, D), lambda i: (1, i, 0)),
        pl.BlockSpec((_R, D), lambda i: (i, 0)),
        pl.BlockSpec((_R, 1), lambda i: (i, 0)),
        pl.BlockSpec((D,), lambda i: (0,)),
    ] + [pl.BlockSpec(W.shape, lambda i: (0, 0)) for W in Ws]
    out_specs = [pl.BlockSpec((_R, W.shape[1]), lambda i: (i, 0)) for W in Ws]
    out_shape = [jax.ShapeDtypeStruct((NP, W.shape[1]), jnp.float32) for W in Ws]
    res = pl.pallas_call(
        body,
        grid=(NP // _R,),
        in_specs=in_specs,
        out_specs=out_specs,
        out_shape=out_shape,
    )(s, s, hd, dinv, b, *Ws)
    return res


_RF = 2000


def _combine_final(s, hd, dinv, b, out_w=None):
    D = hd.shape[1]
    out_w = D if out_w is None else out_w

    def body(s0_ref, s1_ref, hd_ref, v_ref, b_ref, o_ref):
        t = v_ref[...] * (s0_ref[0] + s1_ref[0] + hd_ref[...]) + b_ref[...]
        o_ref[...] = jnp.maximum(t[:, :out_w], 0.0)

    return pl.pallas_call(
        body,
        grid=(N // _RF,),
        in_specs=[
            pl.BlockSpec((1, _RF, D), lambda i: (0, i, 0)),
            pl.BlockSpec((1, _RF, D), lambda i: (1, i, 0)),
            pl.BlockSpec((_RF, D), lambda i: (i, 0)),
            pl.BlockSpec((_RF, 1), lambda i: (i, 0)),
            pl.BlockSpec((D,), lambda i: (0,)),
        ],
        out_specs=pl.BlockSpec((_RF, out_w), lambda i: (i, 0)),
        out_shape=jax.ShapeDtypeStruct((N, out_w), jnp.float32),
    )(s, s, hd, dinv, b)


def _gram(hs):
    BI, BJ = 1024, 1280

    def body(a_ref, b_ref, o_ref):
        o_ref[...] = lax.dot_general(
            a_ref[...], b_ref[...],
            dimension_numbers=(((1,), (1,)), ((), ())),
            preferred_element_type=jnp.float32)

    return pl.pallas_call(
        body,
        grid=(pl.cdiv(N, BI), pl.cdiv(N, BJ)),
        in_specs=[
            pl.BlockSpec((BI, 64), lambda i, j: (i, 0)),
            pl.BlockSpec((BJ, 64), lambda i, j: (j, 0)),
        ],
        out_specs=pl.BlockSpec((BI, BJ), lambda i, j: (i, j)),
        out_shape=jax.ShapeDtypeStruct((N, N), jnp.float32),
    )(hs, hs)


def kernel(x, edge_index, W1, b1, W2, b2, W3, b3, W4, b4, W5, b5):
    pad = EPAD - E
    padidx = (N + (jnp.arange(pad, dtype=jnp.int32) % (NP - N))).astype(jnp.int32)
    src2d = jnp.concatenate([edge_index[0], padidx]).reshape(NC * NS, CPT, CHUNK)
    dst2d = jnp.concatenate([edge_index[1], padidx]).reshape(NC * NS, CPT, CHUNK)
    xp = jnp.pad(x, ((0, NP - N), (0, 0)))
    W2p = jnp.pad(W2, ((0, 0), (0, 64)))
    b2p = jnp.pad(b2, (0, 64))
    W3p = jnp.pad(W3, ((0, 64), (0, 0)))
    W5p = jnp.pad(W5, ((0, 64), (0, 64)))
    b5p = jnp.pad(b5, (0, 64))

    degp = _deg_partials(dst2d)
    dinv = _dinv_from_deg(degp)

    hd1 = _mm_scale(xp, W1, dinv)
    s1 = _scatter(hd1, src2d, dst2d)
    (hd2,) = _combine_mm(s1, hd1, dinv, b1, [W2p])
    s2 = _scatter(hd2, src2d, dst2d)
    hd3, hd5 = _combine_mm(s2, hd2, dinv, b2p, [W3p, W5p])
    s3 = _scatter(hd3, src2d, dst2d)
    (hd4,) = _combine_mm(s3, hd3, dinv, b3, [W4])
    s5 = _scatter(hd5, src2d, dst2d)
    s4 = _scatter(hd4, src2d, dst2d)
    x_hat = _combine_final(s4, hd4, dinv, b4)
    hs = _combine_final(s5, hd5, dinv, b5p, out_w=64)
    a_hat = _gram(hs)
    return (x_hat, a_hat)


# speedup vs baseline: 21.0061x; 1.4635x over previous
import functools

import jax
import jax.numpy as jnp
from jax import lax
from jax.experimental import pallas as pl
from jax.experimental.pallas import tpu as pltpu
from jax.experimental.pallas import tpu_sc as plsc

N = 10000
NP = 10240
E = 320000
NC = 2
NS = 16
CHUNK = 128
CPT = 79
EPAD = NC * NS * CPT * CHUNK
ROWS_PT = NP // NS
DEGW = 16
D = 128


def _sc_mesh():
    return plsc.VectorSubcoreMesh(core_axis_name="c", subcore_axis_name="s",
                                  num_cores=NC, num_subcores=NS)


def _deg_partials(dst3d):
    @functools.partial(
        pl.kernel,
        mesh=_sc_mesh(),
        out_type=jax.ShapeDtypeStruct((NC, NP, DEGW), jnp.float32),
        scratch_types=[
            pltpu.VMEM((CPT, CHUNK), jnp.int32),
            pltpu.VMEM((CHUNK, DEGW), jnp.float32),
            pltpu.VMEM((CHUNK, DEGW), jnp.float32),
            pltpu.VMEM_SHARED((NP, DEGW), jnp.float32),
        ],
    )
    def k(dst_hbm, out_hbm, dst_v, ones_v, zero_v, acc_sh):
        c = lax.axis_index("c")
        s = lax.axis_index("s")
        w = c * NS + s
        one = jnp.ones((16,), jnp.float32)
        zer = jnp.zeros((16,), jnp.float32)

        @pl.loop(0, CHUNK)
        def _(r):
            ones_v[r, pl.ds(0, 16)] = one
            zero_v[r, pl.ds(0, 16)] = zer

        row0 = s * ROWS_PT
        for kk in range(ROWS_PT // CHUNK):
            pltpu.sync_copy(zero_v, acc_sh.at[pl.ds(row0 + kk * CHUNK, CHUNK)])
        pltpu.sync_copy(dst_hbm.at[w], dst_v)
        plsc.subcore_barrier()

        @pl.loop(0, CPT)
        def _(j):
            pltpu.sync_copy(ones_v, acc_sh.at[dst_v.at[j]], add=True)

        plsc.subcore_barrier()
        pltpu.sync_copy(acc_sh.at[pl.ds(row0, ROWS_PT)],
                        out_hbm.at[c, pl.ds(row0, ROWS_PT)])

    return k(dst3d)


def _scatter_kernel():
    @functools.partial(
        pl.kernel,
        mesh=_sc_mesh(),
        out_type=jax.ShapeDtypeStruct((NC, NP, D), jnp.float32),
        scratch_types=[
            pltpu.VMEM((2, CHUNK), jnp.int32),
            pltpu.VMEM((2, CHUNK), jnp.int32),
            pltpu.VMEM((CHUNK, D), jnp.float32),
            pltpu.VMEM((CHUNK, D), jnp.float32),
            pltpu.VMEM_SHARED((NP, D), jnp.float32),
            pltpu.SemaphoreType.DMA,
            pltpu.SemaphoreType.DMA,
            pltpu.SemaphoreType.DMA,
            pltpu.SemaphoreType.DMA,
        ],
    )
    def k(fd_hbm, src_hbm, dst_hbm, out_hbm,
          src_db, dst_db, rows0, rows1, acc_sh, sem0, sem1, semi0, semi1):
        c = lax.axis_index("c")
        s = lax.axis_index("s")
        w = c * NS + s
        zer = jnp.zeros((16,), jnp.float32)

        @pl.loop(0, CHUNK)
        def _(r):
            for cc in range(D // 16):
                rows0[r, pl.ds(cc * 16, 16)] = zer

        row0 = s * ROWS_PT
        for kk in range(ROWS_PT // CHUNK):
            pltpu.sync_copy(rows0, acc_sh.at[pl.ds(row0 + kk * CHUNK, CHUNK)])

        def issue_idx(j, slot, semi):
            pltpu.async_copy(src_hbm.at[w, j], src_db.at[slot], semi)
            pltpu.async_copy(dst_hbm.at[w, j], dst_db.at[slot], semi)

        def wait_idx(j, slot, semi):
            pltpu.make_async_copy(src_hbm.at[w, j], src_db.at[slot], semi).wait()
            pltpu.make_async_copy(dst_hbm.at[w, j], dst_db.at[slot], semi).wait()

        plsc.subcore_barrier()

        issue_idx(0, 0, semi0)
        issue_idx(1, 1, semi1)
        wait_idx(0, 0, semi0)
        pltpu.async_copy(fd_hbm.at[src_db.at[0]], rows0, sem0)

        @pl.loop(0, (CPT - 1) // 2)
        def _(t):
            j0 = 2 * t
            wait_idx(j0 + 1, 1, semi1)
            pltpu.make_async_copy(fd_hbm.at[src_db.at[0]], rows0, sem0).wait()
            pltpu.async_copy(fd_hbm.at[src_db.at[1]], rows1, sem1)
            pltpu.sync_copy(rows0, acc_sh.at[dst_db.at[0]], add=True)
            issue_idx(j0 + 2, 0, semi0)
            pltpu.make_async_copy(fd_hbm.at[src_db.at[1]], rows1, sem1).wait()
            wait_idx(j0 + 2, 0, semi0)
            pltpu.async_copy(fd_hbm.at[src_db.at[0]], rows0, sem0)
            pltpu.sync_copy(rows1, acc_sh.at[dst_db.at[1]], add=True)

            @pl.when(j0 + 3 < CPT)
            def _():
                issue_idx(j0 + 3, 1, semi1)

        pltpu.make_async_copy(fd_hbm.at[src_db.at[0]], rows0, sem0).wait()
        pltpu.sync_copy(rows0, acc_sh.at[dst_db.at[0]], add=True)

        plsc.subcore_barrier()
        pltpu.sync_copy(acc_sh.at[pl.ds(row0, ROWS_PT)],
                        out_hbm.at[c, pl.ds(row0, ROWS_PT)])

    return k


_scatter_cache = {}


def _scatter(fd, src3d, dst3d):
    if "k" not in _scatter_cache:
        _scatter_cache["k"] = _scatter_kernel()
    return _scatter_cache["k"](fd, src3d, dst3d)


_R = 2048


def _dinv_xd(degp, xp):
    def body(d_ref, x_ref, v_ref, xd_ref):
        i = pl.program_id(0)
        total = d_ref[0, :, 0:1] + d_ref[1, :, 0:1] + 1.0
        rowid = i * _R + lax.broadcasted_iota(jnp.int32, (_R, 1), 0)
        v = jnp.where(rowid < N, lax.rsqrt(total), 0.0)
        v_ref[...] = v
        xd_ref[...] = x_ref[...] * v

    return pl.pallas_call(
        body,
        grid=(NP // _R,),
        in_specs=[
            pl.BlockSpec((NC, _R, DEGW), lambda i: (0, i, 0)),
            pl.BlockSpec((_R, D), lambda i: (i, 0)),
        ],
        out_specs=[
            pl.BlockSpec((_R, 1), lambda i: (i, 0)),
            pl.BlockSpec((_R, D), lambda i: (i, 0)),
        ],
        out_shape=[
            jax.ShapeDtypeStruct((NP, 1), jnp.float32),
            jax.ShapeDtypeStruct((NP, D), jnp.float32),
        ],
    )(degp, xp)


def _combine(g, fd, dinv, specs):
    nw = len(specs)

    def body(g0_ref, g1_ref, fd_ref, v_ref, *rest):
        b_refs = rest[:nw]
        w_refs = rest[nw:2 * nw]
        o_refs = rest[2 * nw:]
        v = v_ref[...]
        u = g0_ref[0] + g1_ref[0] + fd_ref[...]
        for (w_ref, b_ref, o_ref, (_, _, scale, out_w, _)) in zip(
                w_refs, b_refs, o_refs, specs):
            t = v * jnp.dot(u, w_ref[...],
                            preferred_element_type=jnp.float32) + b_ref[...]
            t = jnp.maximum(t, 0.0)
            if scale:
                t = t * v
            o_ref[...] = t[:, :out_w]

    in_specs = [
        pl.BlockSpec((1, _R, D), lambda i: (0, i, 0)),
        pl.BlockSpec((1, _R, D), lambda i: (1, i, 0)),
        pl.BlockSpec((_R, D), lambda i: (i, 0)),
        pl.BlockSpec((_R, 1), lambda i: (i, 0)),
    ]
    in_specs += [pl.BlockSpec(sp[1].shape, lambda i: (0,)) for sp in specs]
    in_specs += [pl.BlockSpec(sp[0].shape, lambda i: (0, 0)) for sp in specs]
    out_specs = [pl.BlockSpec((_R, sp[3]), lambda i: (i, 0)) for sp in specs]
    out_shape = [jax.ShapeDtypeStruct((sp[4], sp[3]), jnp.float32)
                 for sp in specs]
    res = pl.pallas_call(
        body,
        grid=(NP // _R,),
        in_specs=in_specs,
        out_specs=out_specs,
        out_shape=out_shape,
    )(g, g, fd, dinv,
      *[sp[1] for sp in specs], *[sp[0] for sp in specs])
    return res


def _gram(hs):
    BI, BJ = 1024, 1280

    def body(a_ref, b_ref, o_ref):
        o_ref[...] = lax.dot_general(
            a_ref[...], b_ref[...],
            dimension_numbers=(((1,), (1,)), ((), ())),
            preferred_element_type=jnp.float32)

    return pl.pallas_call(
        body,
        grid=(pl.cdiv(N, BI), pl.cdiv(N, BJ)),
        in_specs=[
            pl.BlockSpec((BI, 64), lambda i, j: (i, 0)),
            pl.BlockSpec((BJ, 64), lambda i, j: (j, 0)),
        ],
        out_specs=pl.BlockSpec((BI, BJ), lambda i, j: (i, j)),
        out_shape=jax.ShapeDtypeStruct((N, N), jnp.float32),
    )(hs, hs)


def kernel(x, edge_index, W1, b1, W2, b2, W3, b3, W4, b4, W5, b5):
    pad = EPAD - E
    padidx = (N + (jnp.arange(pad, dtype=jnp.int32) % (NP - N))).astype(jnp.int32)
    src3d = jnp.concatenate([edge_index[0], padidx]).reshape(NC * NS, CPT, CHUNK)
    dst3d = jnp.concatenate([edge_index[1], padidx]).reshape(NC * NS, CPT, CHUNK)
    xp = jnp.pad(x, ((0, NP - N), (0, 0)))
    W2p = jnp.pad(W2, ((0, 0), (0, 64)))
    b2p = jnp.pad(b2, (0, 64))
    W3p = jnp.pad(W3, ((0, 64), (0, 0)))
    W5p = jnp.pad(W5, ((0, 64), (0, 64)))
    b5p = jnp.pad(b5, (0, 64))

    degp = _deg_partials(dst3d)
    dinv, xd = _dinv_xd(degp, xp)

    gx = _scatter(xd, src3d, dst3d)
    (hd,) = _combine(gx, xd, dinv, [(W1, b1, True, D, NP)])
    gh = _scatter(hd, src3d, dst3d)
    (zd,) = _combine(gh, hd, dinv, [(W2p, b2p, True, D, NP)])
    gz = _scatter(zd, src3d, dst3d)
    a1d, hs = _combine(gz, zd, dinv, [(W3p, b3, True, D, NP),
                                      (W5p, b5p, False, 64, N)])
    ga = _scatter(a1d, src3d, dst3d)
    (x_hat,) = _combine(ga, a1d, dinv, [(W4, b4, False, D, N)])
    a_hat = _gram(hs)
    return (x_hat, a_hat)


# speedup vs baseline: 21.7945x; 1.0375x over previous
import functools

import jax
import jax.numpy as jnp
from jax import lax
from jax.experimental import pallas as pl
from jax.experimental.pallas import tpu as pltpu
from jax.experimental.pallas import tpu_sc as plsc

N = 10000
NP = 10240
E = 320000
NC = 2
NS = 16
CHUNK = 128
CPT = 79
EPAD = NC * NS * CPT * CHUNK
ROWS_PT = NP // NS
DEGW = 16
D = 128


def _sc_mesh():
    return plsc.VectorSubcoreMesh(core_axis_name="c", subcore_axis_name="s",
                                  num_cores=NC, num_subcores=NS)


def _deg_partials(dst3d):
    @functools.partial(
        pl.kernel,
        mesh=_sc_mesh(),
        out_type=jax.ShapeDtypeStruct((NC, NP, DEGW), jnp.float32),
        scratch_types=[
            pltpu.VMEM((CPT, CHUNK), jnp.int32),
            pltpu.VMEM((CHUNK, DEGW), jnp.float32),
            pltpu.VMEM((CHUNK, DEGW), jnp.float32),
            pltpu.VMEM_SHARED((NP, DEGW), jnp.float32),
        ],
    )
    def k(dst_hbm, out_hbm, dst_v, ones_v, zero_v, acc_sh):
        c = lax.axis_index("c")
        s = lax.axis_index("s")
        w = c * NS + s
        one = jnp.ones((16,), jnp.float32)
        zer = jnp.zeros((16,), jnp.float32)

        @pl.loop(0, CHUNK)
        def _(r):
            ones_v[r, pl.ds(0, 16)] = one
            zero_v[r, pl.ds(0, 16)] = zer

        row0 = s * ROWS_PT
        for kk in range(ROWS_PT // CHUNK):
            pltpu.sync_copy(zero_v, acc_sh.at[pl.ds(row0 + kk * CHUNK, CHUNK)])
        pltpu.sync_copy(dst_hbm.at[w], dst_v)
        plsc.subcore_barrier()

        @pl.loop(0, CPT)
        def _(j):
            pltpu.sync_copy(ones_v, acc_sh.at[dst_v.at[j]], add=True)

        plsc.subcore_barrier()
        pltpu.sync_copy(acc_sh.at[pl.ds(row0, ROWS_PT)],
                        out_hbm.at[c, pl.ds(row0, ROWS_PT)])

    return k(dst3d)


def _scatter_kernel(D, tc_tiling=True):
    @functools.partial(
        pl.kernel,
        mesh=_sc_mesh(),
        out_type=jax.ShapeDtypeStruct((NC, NP, D), jnp.float32),
        compiler_params=pltpu.CompilerParams(use_tc_tiling_on_sc=tc_tiling),
        scratch_types=[
            pltpu.VMEM((2, CHUNK), jnp.int32),
            pltpu.VMEM((2, CHUNK), jnp.int32),
            pltpu.VMEM((CHUNK, D), jnp.float32),
            pltpu.VMEM((CHUNK, D), jnp.float32),
            pltpu.VMEM_SHARED((NP, D), jnp.float32),
            pltpu.SemaphoreType.DMA,
            pltpu.SemaphoreType.DMA,
            pltpu.SemaphoreType.DMA,
            pltpu.SemaphoreType.DMA,
        ],
    )
    def k(fd_hbm, src_hbm, dst_hbm, out_hbm,
          src_db, dst_db, rows0, rows1, acc_sh, sem0, sem1, semi0, semi1):
        c = lax.axis_index("c")
        s = lax.axis_index("s")
        w = c * NS + s
        zer = jnp.zeros((16,), jnp.float32)

        @pl.loop(0, CHUNK)
        def _(r):
            for cc in range(D // 16):
                rows0[r, pl.ds(cc * 16, 16)] = zer

        row0 = s * ROWS_PT
        for kk in range(ROWS_PT // CHUNK):
            pltpu.sync_copy(rows0, acc_sh.at[pl.ds(row0 + kk * CHUNK, CHUNK)])

        def issue_idx(j, slot, semi):
            pltpu.async_copy(src_hbm.at[w, j], src_db.at[slot], semi)
            pltpu.async_copy(dst_hbm.at[w, j], dst_db.at[slot], semi)

        def wait_idx(j, slot, semi):
            pltpu.make_async_copy(src_hbm.at[w, j], src_db.at[slot], semi).wait()
            pltpu.make_async_copy(dst_hbm.at[w, j], dst_db.at[slot], semi).wait()

        plsc.subcore_barrier()

        issue_idx(0, 0, semi0)
        issue_idx(1, 1, semi1)
        wait_idx(0, 0, semi0)
        pltpu.async_copy(fd_hbm.at[src_db.at[0]], rows0, sem0)

        @pl.loop(0, (CPT - 1) // 2)
        def _(t):
            j0 = 2 * t
            wait_idx(j0 + 1, 1, semi1)
            pltpu.make_async_copy(fd_hbm.at[src_db.at[0]], rows0, sem0).wait()
            pltpu.async_copy(fd_hbm.at[src_db.at[1]], rows1, sem1)
            pltpu.sync_copy(rows0, acc_sh.at[dst_db.at[0]], add=True)
            issue_idx(j0 + 2, 0, semi0)
            pltpu.make_async_copy(fd_hbm.at[src_db.at[1]], rows1, sem1).wait()
            wait_idx(j0 + 2, 0, semi0)
            pltpu.async_copy(fd_hbm.at[src_db.at[0]], rows0, sem0)
            pltpu.sync_copy(rows1, acc_sh.at[dst_db.at[1]], add=True)

            @pl.when(j0 + 3 < CPT)
            def _():
                issue_idx(j0 + 3, 1, semi1)

        pltpu.make_async_copy(fd_hbm.at[src_db.at[0]], rows0, sem0).wait()
        pltpu.sync_copy(rows0, acc_sh.at[dst_db.at[0]], add=True)

        plsc.subcore_barrier()
        pltpu.sync_copy(acc_sh.at[pl.ds(row0, ROWS_PT)],
                        out_hbm.at[c, pl.ds(row0, ROWS_PT)])

    return k


_scatter_cache = {}


def _scatter(fd, src3d, dst3d):
    D = fd.shape[1]
    if D not in _scatter_cache:
        _scatter_cache[D] = _scatter_kernel(D, tc_tiling=(D % 128 == 0))
    return _scatter_cache[D](fd, src3d, dst3d)


_R = 2048


def _dinv_xd(degp, xp):
    def body(d_ref, x_ref, v_ref, xd_ref):
        i = pl.program_id(0)
        total = d_ref[0, :, 0:1] + d_ref[1, :, 0:1] + 1.0
        rowid = i * _R + lax.broadcasted_iota(jnp.int32, (_R, 1), 0)
        v = jnp.where(rowid < N, lax.rsqrt(total), 0.0)
        v_ref[...] = v
        xd_ref[...] = x_ref[...] * v

    return pl.pallas_call(
        body,
        grid=(NP // _R,),
        in_specs=[
            pl.BlockSpec((NC, _R, DEGW), lambda i: (0, i, 0)),
            pl.BlockSpec((_R, D), lambda i: (i, 0)),
        ],
        out_specs=[
            pl.BlockSpec((_R, 1), lambda i: (i, 0)),
            pl.BlockSpec((_R, D), lambda i: (i, 0)),
        ],
        out_shape=[
            jax.ShapeDtypeStruct((NP, 1), jnp.float32),
            jax.ShapeDtypeStruct((NP, D), jnp.float32),
        ],
    )(degp, xp)


def _combine(g, fd, dinv, specs):
    nw = len(specs)
    Din = fd.shape[1]

    def body(g0_ref, g1_ref, fd_ref, v_ref, *rest):
        b_refs = rest[:nw]
        w_refs = rest[nw:2 * nw]
        o_refs = rest[2 * nw:]
        v = v_ref[...]
        u = g0_ref[0] + g1_ref[0] + fd_ref[...]
        for (w_ref, b_ref, o_ref, (_, _, scale, out_w, _)) in zip(
                w_refs, b_refs, o_refs, specs):
            t = v * jnp.dot(u, w_ref[...],
                            preferred_element_type=jnp.float32) + b_ref[...]
            t = jnp.maximum(t, 0.0)
            if scale:
                t = t * v
            o_ref[...] = t[:, :out_w]

    in_specs = [
        pl.BlockSpec((1, _R, Din), lambda i: (0, i, 0)),
        pl.BlockSpec((1, _R, Din), lambda i: (1, i, 0)),
        pl.BlockSpec((_R, Din), lambda i: (i, 0)),
        pl.BlockSpec((_R, 1), lambda i: (i, 0)),
    ]
    in_specs += [pl.BlockSpec(sp[1].shape, lambda i: (0,)) for sp in specs]
    in_specs += [pl.BlockSpec(sp[0].shape, lambda i: (0, 0)) for sp in specs]
    out_specs = [pl.BlockSpec((_R, sp[3]), lambda i: (i, 0)) for sp in specs]
    out_shape = [jax.ShapeDtypeStruct((sp[4], sp[3]), jnp.float32)
                 for sp in specs]
    res = pl.pallas_call(
        body,
        grid=(NP // _R,),
        in_specs=in_specs,
        out_specs=out_specs,
        out_shape=out_shape,
    )(g, g, fd, dinv,
      *[sp[1] for sp in specs], *[sp[0] for sp in specs])
    return res


def _gram(hs):
    BI, BJ = 1024, 1280

    def body(a_ref, b_ref, o_ref):
        o_ref[...] = lax.dot_general(
            a_ref[...], b_ref[...],
            dimension_numbers=(((1,), (1,)), ((), ())),
            preferred_element_type=jnp.float32)

    return pl.pallas_call(
        body,
        grid=(pl.cdiv(N, BI), pl.cdiv(N, BJ)),
        in_specs=[
            pl.BlockSpec((BI, 64), lambda i, j: (i, 0)),
            pl.BlockSpec((BJ, 64), lambda i, j: (j, 0)),
        ],
        out_specs=pl.BlockSpec((BI, BJ), lambda i, j: (i, j)),
        out_shape=jax.ShapeDtypeStruct((N, N), jnp.float32),
    )(hs, hs)


def kernel(x, edge_index, W1, b1, W2, b2, W3, b3, W4, b4, W5, b5):
    pad = EPAD - E
    padidx = (N + (jnp.arange(pad, dtype=jnp.int32) % (NP - N))).astype(jnp.int32)
    src3d = jnp.concatenate([edge_index[0], padidx]).reshape(NC * NS, CPT, CHUNK)
    dst3d = jnp.concatenate([edge_index[1], padidx]).reshape(NC * NS, CPT, CHUNK)
    xp = jnp.pad(x, ((0, NP - N), (0, 0)))

    degp = _deg_partials(dst3d)
    dinv, xd = _dinv_xd(degp, xp)

    gx = _scatter(xd, src3d, dst3d)
    (hd,) = _combine(gx, xd, dinv, [(W1, b1, True, D, NP)])
    gh = _scatter(hd, src3d, dst3d)
    (zd,) = _combine(gh, hd, dinv, [(W2, b2, True, 64, NP)])
    gz = _scatter(zd, src3d, dst3d)
    a1d, hs = _combine(gz, zd, dinv, [(W3, b3, True, D, NP),
                                      (W5, b5, False, 64, N)])
    ga = _scatter(a1d, src3d, dst3d)
    (x_hat,) = _combine(ga, a1d, dinv, [(W4, b4, False, D, N)])
    a_hat = _gram(hs)
    return (x_hat, a_hat)


# speedup vs baseline: 21.8314x; 1.0017x over previous
import functools

import jax
import jax.numpy as jnp
from jax import lax
from jax.experimental import pallas as pl
from jax.experimental.pallas import tpu as pltpu
from jax.experimental.pallas import tpu_sc as plsc

N = 10000
NP = 10240
E = 320000
NC = 2
NS = 16
CHUNK = 128
CPT = 79
EPAD = NC * NS * CPT * CHUNK
ROWS_PT = NP // NS
DEGW = 16
D = 128


def _sc_mesh():
    return plsc.VectorSubcoreMesh(core_axis_name="c", subcore_axis_name="s",
                                  num_cores=NC, num_subcores=NS)


def _deg_partials(dst3d):
    @functools.partial(
        pl.kernel,
        mesh=_sc_mesh(),
        out_type=jax.ShapeDtypeStruct((NC, NP, DEGW), jnp.float32),
        scratch_types=[
            pltpu.VMEM((CPT, CHUNK), jnp.int32),
            pltpu.VMEM((CHUNK, DEGW), jnp.float32),
            pltpu.VMEM((CHUNK, DEGW), jnp.float32),
            pltpu.VMEM_SHARED((NP, DEGW), jnp.float32),
        ],
    )
    def k(dst_hbm, out_hbm, dst_v, ones_v, zero_v, acc_sh):
        c = lax.axis_index("c")
        s = lax.axis_index("s")
        w = c * NS + s
        one = jnp.ones((16,), jnp.float32)
        zer = jnp.zeros((16,), jnp.float32)

        @pl.loop(0, CHUNK)
        def _(r):
            ones_v[r, pl.ds(0, 16)] = one
            zero_v[r, pl.ds(0, 16)] = zer

        row0 = s * ROWS_PT
        for kk in range(ROWS_PT // CHUNK):
            pltpu.sync_copy(zero_v, acc_sh.at[pl.ds(row0 + kk * CHUNK, CHUNK)])
        pltpu.sync_copy(dst_hbm.at[w], dst_v)
        plsc.subcore_barrier()

        @pl.loop(0, CPT)
        def _(j):
            pltpu.sync_copy(ones_v, acc_sh.at[dst_v.at[j]], add=True)

        plsc.subcore_barrier()
        pltpu.sync_copy(acc_sh.at[pl.ds(row0, ROWS_PT)],
                        out_hbm.at[c, pl.ds(row0, ROWS_PT)])

    return k(dst3d)


def _scatter_kernel(D, tc_tiling=True):
    @functools.partial(
        pl.kernel,
        mesh=_sc_mesh(),
        out_type=jax.ShapeDtypeStruct((NC, NP, D), jnp.float32),
        compiler_params=pltpu.CompilerParams(use_tc_tiling_on_sc=tc_tiling),
        scratch_types=[
            pltpu.VMEM((2, CHUNK), jnp.int32),
            pltpu.VMEM((2, CHUNK), jnp.int32),
            pltpu.VMEM((CHUNK, D), jnp.float32),
            pltpu.VMEM((CHUNK, D), jnp.float32),
            pltpu.VMEM_SHARED((NP, D), jnp.float32),
            pltpu.SemaphoreType.DMA,
            pltpu.SemaphoreType.DMA,
            pltpu.SemaphoreType.DMA,
            pltpu.SemaphoreType.DMA,
        ],
    )
    def k(fd_hbm, src_hbm, dst_hbm, out_hbm,
          src_db, dst_db, rows0, rows1, acc_sh, sem0, sem1, semi0, semi1):
        c = lax.axis_index("c")
        s = lax.axis_index("s")
        w = c * NS + s
        zer = jnp.zeros((16,), jnp.float32)

        @pl.loop(0, CHUNK)
        def _(r):
            for cc in range(D // 16):
                rows0[r, pl.ds(cc * 16, 16)] = zer

        row0 = s * ROWS_PT
        for kk in range(ROWS_PT // CHUNK):
            pltpu.sync_copy(rows0, acc_sh.at[pl.ds(row0 + kk * CHUNK, CHUNK)])

        def issue_idx(j, slot, semi):
            pltpu.async_copy(src_hbm.at[w, j], src_db.at[slot], semi)
            pltpu.async_copy(dst_hbm.at[w, j], dst_db.at[slot], semi)

        def wait_idx(j, slot, semi):
            pltpu.make_async_copy(src_hbm.at[w, j], src_db.at[slot], semi).wait()
            pltpu.make_async_copy(dst_hbm.at[w, j], dst_db.at[slot], semi).wait()

        plsc.subcore_barrier()

        issue_idx(0, 0, semi0)
        issue_idx(1, 1, semi1)
        wait_idx(0, 0, semi0)
        pltpu.async_copy(fd_hbm.at[src_db.at[0]], rows0, sem0)

        @pl.loop(0, (CPT - 1) // 2)
        def _(t):
            j0 = 2 * t
            wait_idx(j0 + 1, 1, semi1)
            pltpu.make_async_copy(fd_hbm.at[src_db.at[0]], rows0, sem0).wait()
            pltpu.async_copy(fd_hbm.at[src_db.at[1]], rows1, sem1)
            pltpu.sync_copy(rows0, acc_sh.at[dst_db.at[0]], add=True)
            issue_idx(j0 + 2, 0, semi0)
            pltpu.make_async_copy(fd_hbm.at[src_db.at[1]], rows1, sem1).wait()
            wait_idx(j0 + 2, 0, semi0)
            pltpu.async_copy(fd_hbm.at[src_db.at[0]], rows0, sem0)
            pltpu.sync_copy(rows1, acc_sh.at[dst_db.at[1]], add=True)

            @pl.when(j0 + 3 < CPT)
            def _():
                issue_idx(j0 + 3, 1, semi1)

        pltpu.make_async_copy(fd_hbm.at[src_db.at[0]], rows0, sem0).wait()
        pltpu.sync_copy(rows0, acc_sh.at[dst_db.at[0]], add=True)

        plsc.subcore_barrier()
        pltpu.sync_copy(acc_sh.at[pl.ds(row0, ROWS_PT)],
                        out_hbm.at[c, pl.ds(row0, ROWS_PT)])

    return k


_scatter_cache = {}


def _scatter(fd, src3d, dst3d):
    D = fd.shape[1]
    if D not in _scatter_cache:
        _scatter_cache[D] = _scatter_kernel(D, tc_tiling=(D % 128 == 0))
    return _scatter_cache[D](fd, src3d, dst3d)


_R = 2048


def _dinv_xd(degp, xp):
    def body(d_ref, x_ref, v_ref, xd_ref):
        i = pl.program_id(0)
        total = d_ref[0, :, 0:1] + d_ref[1, :, 0:1] + 1.0
        rowid = i * _R + lax.broadcasted_iota(jnp.int32, (_R, 1), 0)
        v = jnp.where(rowid < N, lax.rsqrt(total), 0.0)
        v_ref[...] = v
        xd_ref[...] = x_ref[...] * v

    return pl.pallas_call(
        body,
        grid=(NP // _R,),
        in_specs=[
            pl.BlockSpec((NC, _R, DEGW), lambda i: (0, i, 0)),
            pl.BlockSpec((_R, D), lambda i: (i, 0)),
        ],
        out_specs=[
            pl.BlockSpec((_R, 1), lambda i: (i, 0)),
            pl.BlockSpec((_R, D), lambda i: (i, 0)),
        ],
        out_shape=[
            jax.ShapeDtypeStruct((NP, 1), jnp.float32),
            jax.ShapeDtypeStruct((NP, D), jnp.float32),
        ],
    )(degp, xp)


def _combine(g, fd, dinv, specs):
    nw = len(specs)
    Din = fd.shape[1]

    def body(g0_ref, g1_ref, fd_ref, v_ref, *rest):
        b_refs = rest[:nw]
        w_refs = rest[nw:2 * nw]
        o_refs = rest[2 * nw:]
        v = v_ref[...]
        u = g0_ref[0] + g1_ref[0] + fd_ref[...]
        for (w_ref, b_ref, o_ref, (_, _, scale, out_w, _)) in zip(
                w_refs, b_refs, o_refs, specs):
            t = v * jnp.dot(u, w_ref[...],
                            preferred_element_type=jnp.float32) + b_ref[...]
            t = jnp.maximum(t, 0.0)
            if scale:
                t = t * v
            o_ref[...] = t[:, :out_w]

    in_specs = [
        pl.BlockSpec((1, _R, Din), lambda i: (0, i, 0)),
        pl.BlockSpec((1, _R, Din), lambda i: (1, i, 0)),
        pl.BlockSpec((_R, Din), lambda i: (i, 0)),
        pl.BlockSpec((_R, 1), lambda i: (i, 0)),
    ]
    in_specs += [pl.BlockSpec(sp[1].shape, lambda i: (0,)) for sp in specs]
    in_specs += [pl.BlockSpec(sp[0].shape, lambda i: (0, 0)) for sp in specs]
    out_specs = [pl.BlockSpec((_R, sp[3]), lambda i: (i, 0)) for sp in specs]
    out_shape = [jax.ShapeDtypeStruct((sp[4], sp[3]), jnp.float32)
                 for sp in specs]
    res = pl.pallas_call(
        body,
        grid=(NP // _R,),
        in_specs=in_specs,
        out_specs=out_specs,
        out_shape=out_shape,
    )(g, g, fd, dinv,
      *[sp[1] for sp in specs], *[sp[0] for sp in specs])
    return res


def _gram(hs):
    BI, BJ = 1024, 1280

    def body(a_ref, b_ref, o_ref):
        o_ref[...] = lax.dot_general(
            a_ref[...].astype(jnp.bfloat16), b_ref[...].astype(jnp.bfloat16),
            dimension_numbers=(((1,), (1,)), ((), ())),
            preferred_element_type=jnp.float32)

    return pl.pallas_call(
        body,
        grid=(pl.cdiv(N, BI), pl.cdiv(N, BJ)),
        in_specs=[
            pl.BlockSpec((BI, 64), lambda i, j: (i, 0)),
            pl.BlockSpec((BJ, 64), lambda i, j: (j, 0)),
        ],
        out_specs=pl.BlockSpec((BI, BJ), lambda i, j: (i, j)),
        out_shape=jax.ShapeDtypeStruct((N, N), jnp.float32),
    )(hs, hs)


def kernel(x, edge_index, W1, b1, W2, b2, W3, b3, W4, b4, W5, b5):
    pad = EPAD - E
    padidx = (N + (jnp.arange(pad, dtype=jnp.int32) % (NP - N))).astype(jnp.int32)
    src3d = jnp.concatenate([edge_index[0], padidx]).reshape(NC * NS, CPT, CHUNK)
    dst3d = jnp.concatenate([edge_index[1], padidx]).reshape(NC * NS, CPT, CHUNK)
    xp = jnp.pad(x, ((0, NP - N), (0, 0)))

    degp = _deg_partials(dst3d)
    dinv, xd = _dinv_xd(degp, xp)

    gx = _scatter(xd, src3d, dst3d)
    (hd,) = _combine(gx, xd, dinv, [(W1, b1, True, D, NP)])
    gh = _scatter(hd, src3d, dst3d)
    (zd,) = _combine(gh, hd, dinv, [(W2, b2, True, 64, NP)])
    gz = _scatter(zd, src3d, dst3d)
    a1d, hs = _combine(gz, zd, dinv, [(W3, b3, True, D, NP),
                                      (W5, b5, False, 64, N)])
    ga = _scatter(a1d, src3d, dst3d)
    (x_hat,) = _combine(ga, a1d, dinv, [(W4, b4, False, D, N)])
    a_hat = _gram(hs)
    return (x_hat, a_hat)


# speedup vs baseline: 24.2777x; 1.1121x over previous
import functools

import jax
import jax.numpy as jnp
from jax import lax
from jax.experimental import pallas as pl
from jax.experimental.pallas import tpu as pltpu
from jax.experimental.pallas import tpu_sc as plsc

N = 10000
NP = 10240
E = 320000
NC = 2
NS = 16
CHUNK = 128
CPT = 79
EPAD = NC * NS * CPT * CHUNK
ROWS_PT = NP // NS
DEGW = 16
D = 128


def _sc_mesh():
    return plsc.VectorSubcoreMesh(core_axis_name="c", subcore_axis_name="s",
                                  num_cores=NC, num_subcores=NS)


def _deg_partials(dst3d):
    @functools.partial(
        pl.kernel,
        mesh=_sc_mesh(),
        out_type=jax.ShapeDtypeStruct((NC, NP, DEGW), jnp.float32),
        scratch_types=[
            pltpu.VMEM((CPT, CHUNK), jnp.int32),
            pltpu.VMEM((CHUNK, DEGW), jnp.float32),
            pltpu.VMEM((CHUNK, DEGW), jnp.float32),
            pltpu.VMEM_SHARED((NP, DEGW), jnp.float32),
        ],
    )
    def k(dst_hbm, out_hbm, dst_v, ones_v, zero_v, acc_sh):
        c = lax.axis_index("c")
        s = lax.axis_index("s")
        w = c * NS + s
        one = jnp.ones((16,), jnp.float32)
        zer = jnp.zeros((16,), jnp.float32)

        @pl.loop(0, CHUNK)
        def _(r):
            ones_v[r, pl.ds(0, 16)] = one
            zero_v[r, pl.ds(0, 16)] = zer

        row0 = s * ROWS_PT
        for kk in range(ROWS_PT // CHUNK):
            pltpu.sync_copy(zero_v, acc_sh.at[pl.ds(row0 + kk * CHUNK, CHUNK)])
        pltpu.sync_copy(dst_hbm.at[w], dst_v)
        plsc.subcore_barrier()

        @pl.loop(0, CPT)
        def _(j):
            pltpu.sync_copy(ones_v, acc_sh.at[dst_v.at[j]], add=True)

        plsc.subcore_barrier()
        pltpu.sync_copy(acc_sh.at[pl.ds(row0, ROWS_PT)],
                        out_hbm.at[c, pl.ds(row0, ROWS_PT)])

    return k(dst3d)


def _scatter_kernel(D, tc_tiling=True):
    @functools.partial(
        pl.kernel,
        mesh=_sc_mesh(),
        out_type=jax.ShapeDtypeStruct((NC, NP, D), jnp.float32),
        compiler_params=pltpu.CompilerParams(use_tc_tiling_on_sc=tc_tiling),
        scratch_types=[
            pltpu.VMEM((3, CHUNK), jnp.int32),
            pltpu.VMEM((3, CHUNK), jnp.int32),
            pltpu.VMEM((2, CHUNK, D), jnp.float32),
            pltpu.VMEM_SHARED((NP, D), jnp.float32),
            pltpu.SemaphoreType.DMA,
            pltpu.SemaphoreType.DMA,
            pltpu.SemaphoreType.DMA,
            pltpu.SemaphoreType.DMA,
            pltpu.SemaphoreType.DMA,
            pltpu.SemaphoreType.DMA,
        ],
    )
    def k(fd_hbm, src_hbm, dst_hbm, out_hbm,
          src_db, dst_db, rows, acc_sh,
          semi0, semi1, semi2, semg0, semg1, semS):
        c = lax.axis_index("c")
        s = lax.axis_index("s")
        w = c * NS + s
        zer = jnp.zeros((16,), jnp.float32)
        semi = (semi0, semi1, semi2)
        semg = (semg0, semg1)

        @pl.loop(0, CHUNK)
        def _(r):
            for cc in range(D // 16):
                rows[0, r, pl.ds(cc * 16, 16)] = zer

        row0 = s * ROWS_PT
        for kk in range(ROWS_PT // CHUNK):
            pltpu.sync_copy(rows.at[0],
                            acc_sh.at[pl.ds(row0 + kk * CHUNK, CHUNK)])

        def issue_idx(j, q):
            pltpu.async_copy(src_hbm.at[w, j], src_db.at[q], semi[q])
            pltpu.async_copy(dst_hbm.at[w, j], dst_db.at[q], semi[q])

        def wait_idx(j, q):
            pltpu.make_async_copy(src_hbm.at[w, j], src_db.at[q],
                                  semi[q]).wait()
            pltpu.make_async_copy(dst_hbm.at[w, j], dst_db.at[q],
                                  semi[q]).wait()

        def issue_gather(q, r):
            pltpu.async_copy(fd_hbm.at[src_db.at[q]], rows.at[r], semg[r])

        def wait_gather(q, r):
            pltpu.make_async_copy(fd_hbm.at[src_db.at[q]], rows.at[r],
                                  semg[r]).wait()

        def issue_scatter(q, r):
            pltpu.async_copy(rows.at[r], acc_sh.at[dst_db.at[q]], semS,
                             add=True)

        def wait_scatter(q, r):
            pltpu.make_async_copy(rows.at[r], acc_sh.at[dst_db.at[q]],
                                  semS).wait()

        plsc.subcore_barrier()

        issue_idx(0, 0)
        issue_idx(1, 1)
        wait_idx(0, 0)
        issue_gather(0, 0)

        @pl.loop(0, (CPT - 1) // 6)
        def _(t):
            for k in range(6):
                j = 6 * t + k
                r = k % 2
                rn = (k + 1) % 2
                q = k % 3
                qp = (k - 1) % 3
                qn = (k + 1) % 3
                qi = (k + 2) % 3

                wait_idx(j + 1, qn)

                @pl.when(j >= 1)
                def _(qp=qp, rn=rn):
                    wait_scatter(qp, rn)

                issue_gather(qn, rn)

                @pl.when(j + 2 < CPT)
                def _(j=j, qi=qi):
                    issue_idx(j + 2, qi)

                wait_gather(q, r)
                issue_scatter(q, r)

        wait_scatter(2, 1)
        wait_gather(0, 0)
        issue_scatter(0, 0)
        wait_scatter(0, 0)

        plsc.subcore_barrier()
        pltpu.sync_copy(acc_sh.at[pl.ds(row0, ROWS_PT)],
                        out_hbm.at[c, pl.ds(row0, ROWS_PT)])

    return k


_scatter_cache = {}


def _scatter(fd, src3d, dst3d):
    D = fd.shape[1]
    if D not in _scatter_cache:
        _scatter_cache[D] = _scatter_kernel(D, tc_tiling=(D % 128 == 0))
    return _scatter_cache[D](fd, src3d, dst3d)


_R = 2048


def _dinv_xd(degp, xp):
    def body(d_ref, x_ref, v_ref, xd_ref):
        i = pl.program_id(0)
        total = d_ref[0, :, 0:1] + d_ref[1, :, 0:1] + 1.0
        rowid = i * _R + lax.broadcasted_iota(jnp.int32, (_R, 1), 0)
        v = jnp.where(rowid < N, lax.rsqrt(total), 0.0)
        v_ref[...] = v
        xd_ref[...] = x_ref[...] * v

    return pl.pallas_call(
        body,
        grid=(NP // _R,),
        in_specs=[
            pl.BlockSpec((NC, _R, DEGW), lambda i: (0, i, 0)),
            pl.BlockSpec((_R, D), lambda i: (i, 0)),
        ],
        out_specs=[
            pl.BlockSpec((_R, 1), lambda i: (i, 0)),
            pl.BlockSpec((_R, D), lambda i: (i, 0)),
        ],
        out_shape=[
            jax.ShapeDtypeStruct((NP, 1), jnp.float32),
            jax.ShapeDtypeStruct((NP, D), jnp.float32),
        ],
    )(degp, xp)


def _combine(g, fd, dinv, specs):
    nw = len(specs)
    Din = fd.shape[1]

    def body(g0_ref, g1_ref, fd_ref, v_ref, *rest):
        b_refs = rest[:nw]
        w_refs = rest[nw:2 * nw]
        o_refs = rest[2 * nw:]
        v = v_ref[...]
        u = g0_ref[0] + g1_ref[0] + fd_ref[...]
        for (w_ref, b_ref, o_ref, (_, _, scale, out_w, _)) in zip(
                w_refs, b_refs, o_refs, specs):
            t = v * jnp.dot(u, w_ref[...],
                            preferred_element_type=jnp.float32) + b_ref[...]
            t = jnp.maximum(t, 0.0)
            if scale:
                t = t * v
            o_ref[...] = t[:, :out_w]

    in_specs = [
        pl.BlockSpec((1, _R, Din), lambda i: (0, i, 0)),
        pl.BlockSpec((1, _R, Din), lambda i: (1, i, 0)),
        pl.BlockSpec((_R, Din), lambda i: (i, 0)),
        pl.BlockSpec((_R, 1), lambda i: (i, 0)),
    ]
    in_specs += [pl.BlockSpec(sp[1].shape, lambda i: (0,)) for sp in specs]
    in_specs += [pl.BlockSpec(sp[0].shape, lambda i: (0, 0)) for sp in specs]
    out_specs = [pl.BlockSpec((_R, sp[3]), lambda i: (i, 0)) for sp in specs]
    out_shape = [jax.ShapeDtypeStruct((sp[4], sp[3]), jnp.float32)
                 for sp in specs]
    res = pl.pallas_call(
        body,
        grid=(NP // _R,),
        in_specs=in_specs,
        out_specs=out_specs,
        out_shape=out_shape,
    )(g, g, fd, dinv,
      *[sp[1] for sp in specs], *[sp[0] for sp in specs])
    return res


def _gram(hs):
    BI, BJ = 1024, 1280

    def body(a_ref, b_ref, o_ref):
        o_ref[...] = lax.dot_general(
            a_ref[...].astype(jnp.bfloat16), b_ref[...].astype(jnp.bfloat16),
            dimension_numbers=(((1,), (1,)), ((), ())),
            preferred_element_type=jnp.float32)

    return pl.pallas_call(
        body,
        grid=(pl.cdiv(N, BI), pl.cdiv(N, BJ)),
        in_specs=[
            pl.BlockSpec((BI, 64), lambda i, j: (i, 0)),
            pl.BlockSpec((BJ, 64), lambda i, j: (j, 0)),
        ],
        out_specs=pl.BlockSpec((BI, BJ), lambda i, j: (i, j)),
        out_shape=jax.ShapeDtypeStruct((N, N), jnp.float32),
    )(hs, hs)


def kernel(x, edge_index, W1, b1, W2, b2, W3, b3, W4, b4, W5, b5):
    pad = EPAD - E
    padidx = (N + (jnp.arange(pad, dtype=jnp.int32) % (NP - N))).astype(jnp.int32)
    src3d = jnp.concatenate([edge_index[0], padidx]).reshape(NC * NS, CPT, CHUNK)
    dst3d = jnp.concatenate([edge_index[1], padidx]).reshape(NC * NS, CPT, CHUNK)
    xp = jnp.pad(x, ((0, NP - N), (0, 0)))

    degp = _deg_partials(dst3d)
    dinv, xd = _dinv_xd(degp, xp)

    gx = _scatter(xd, src3d, dst3d)
    (hd,) = _combine(gx, xd, dinv, [(W1, b1, True, D, NP)])
    gh = _scatter(hd, src3d, dst3d)
    (zd,) = _combine(gh, hd, dinv, [(W2, b2, True, 64, NP)])
    gz = _scatter(zd, src3d, dst3d)
    a1d, hs = _combine(gz, zd, dinv, [(W3, b3, True, D, NP),
                                      (W5, b5, False, 64, N)])
    ga = _scatter(a1d, src3d, dst3d)
    (x_hat,) = _combine(ga, a1d, dinv, [(W4, b4, False, D, N)])
    a_hat = _gram(hs)
    return (x_hat, a_hat)


# speedup vs baseline: 24.5005x; 1.0092x over previous
import functools

import jax
import jax.numpy as jnp
from jax import lax
from jax.experimental import pallas as pl
from jax.experimental.pallas import tpu as pltpu
from jax.experimental.pallas import tpu_sc as plsc

N = 10000
NP = 10240
E = 320000
NC = 2
NS = 16
CHUNK = 128
CPT = 79
EPAD = NC * NS * CPT * CHUNK
ROWS_PT = NP // NS
DEGW = 16
D = 128


def _sc_mesh():
    return plsc.VectorSubcoreMesh(core_axis_name="c", subcore_axis_name="s",
                                  num_cores=NC, num_subcores=NS)


def _deg_partials(dst3d):
    @functools.partial(
        pl.kernel,
        mesh=_sc_mesh(),
        out_type=jax.ShapeDtypeStruct((NC, NP, DEGW), jnp.float32),
        scratch_types=[
            pltpu.VMEM((CPT, CHUNK), jnp.int32),
            pltpu.VMEM((CHUNK, DEGW), jnp.float32),
            pltpu.VMEM((CHUNK, DEGW), jnp.float32),
            pltpu.VMEM_SHARED((NP, DEGW), jnp.float32),
        ],
    )
    def k(dst_hbm, out_hbm, dst_v, ones_v, zero_v, acc_sh):
        c = lax.axis_index("c")
        s = lax.axis_index("s")
        w = c * NS + s
        one = jnp.ones((16,), jnp.float32)
        zer = jnp.zeros((16,), jnp.float32)

        @pl.loop(0, CHUNK)
        def _(r):
            ones_v[r, pl.ds(0, 16)] = one
            zero_v[r, pl.ds(0, 16)] = zer

        row0 = s * ROWS_PT
        for kk in range(ROWS_PT // CHUNK):
            pltpu.sync_copy(zero_v, acc_sh.at[pl.ds(row0 + kk * CHUNK, CHUNK)])
        pltpu.sync_copy(dst_hbm.at[w], dst_v)
        plsc.subcore_barrier()

        @pl.loop(0, CPT)
        def _(j):
            pltpu.sync_copy(ones_v, acc_sh.at[dst_v.at[j]], add=True)

        plsc.subcore_barrier()
        pltpu.sync_copy(acc_sh.at[pl.ds(row0, ROWS_PT)],
                        out_hbm.at[c, pl.ds(row0, ROWS_PT)])

    return k(dst3d)


def _scatter_kernel(D, tc_tiling=True):
    @functools.partial(
        pl.kernel,
        mesh=_sc_mesh(),
        out_type=jax.ShapeDtypeStruct((NC, NP, D), jnp.float32),
        compiler_params=pltpu.CompilerParams(use_tc_tiling_on_sc=tc_tiling),
        scratch_types=[
            pltpu.VMEM((3, CHUNK), jnp.int32),
            pltpu.VMEM((3, CHUNK), jnp.int32),
            pltpu.VMEM((2, CHUNK, D), jnp.float32),
            pltpu.VMEM_SHARED((NP, D), jnp.float32),
            pltpu.SemaphoreType.DMA,
            pltpu.SemaphoreType.DMA,
            pltpu.SemaphoreType.DMA,
            pltpu.SemaphoreType.DMA,
            pltpu.SemaphoreType.DMA,
            pltpu.SemaphoreType.DMA,
        ],
    )
    def k(fd_hbm, src_hbm, dst_hbm, out_hbm,
          src_db, dst_db, rows, acc_sh,
          semi0, semi1, semi2, semg0, semg1, semS):
        c = lax.axis_index("c")
        s = lax.axis_index("s")
        w = c * NS + s
        zer = jnp.zeros((16,), jnp.float32)
        semi = (semi0, semi1, semi2)
        semg = (semg0, semg1)

        @pl.loop(0, CHUNK)
        def _(r):
            for cc in range(D // 16):
                rows[0, r, pl.ds(cc * 16, 16)] = zer

        row0 = s * ROWS_PT
        for kk in range(ROWS_PT // CHUNK):
            pltpu.sync_copy(rows.at[0],
                            acc_sh.at[pl.ds(row0 + kk * CHUNK, CHUNK)])

        def issue_idx(j, q):
            pltpu.async_copy(src_hbm.at[w, j], src_db.at[q], semi[q])
            pltpu.async_copy(dst_hbm.at[w, j], dst_db.at[q], semi[q])

        def wait_idx(j, q):
            pltpu.make_async_copy(src_hbm.at[w, j], src_db.at[q],
                                  semi[q]).wait()
            pltpu.make_async_copy(dst_hbm.at[w, j], dst_db.at[q],
                                  semi[q]).wait()

        def issue_gather(q, r):
            pltpu.async_copy(fd_hbm.at[src_db.at[q]], rows.at[r], semg[r])

        def wait_gather(q, r):
            pltpu.make_async_copy(fd_hbm.at[src_db.at[q]], rows.at[r],
                                  semg[r]).wait()

        def issue_scatter(q, r):
            pltpu.async_copy(rows.at[r], acc_sh.at[dst_db.at[q]], semS,
                             add=True)

        def wait_scatter(q, r):
            pltpu.make_async_copy(rows.at[r], acc_sh.at[dst_db.at[q]],
                                  semS).wait()

        plsc.subcore_barrier()

        issue_idx(0, 0)
        issue_idx(1, 1)
        wait_idx(0, 0)
        issue_gather(0, 0)

        @pl.loop(0, (CPT - 1) // 6)
        def _(t):
            for k in range(6):
                j = 6 * t + k
                r = k % 2
                rn = (k + 1) % 2
                q = k % 3
                qp = (k - 1) % 3
                qn = (k + 1) % 3
                qi = (k + 2) % 3

                wait_idx(j + 1, qn)

                @pl.when(j >= 1)
                def _(qp=qp, rn=rn):
                    wait_scatter(qp, rn)

                issue_gather(qn, rn)

                @pl.when(j + 2 < CPT)
                def _(j=j, qi=qi):
                    issue_idx(j + 2, qi)

                wait_gather(q, r)
                issue_scatter(q, r)

        wait_scatter(2, 1)
        wait_gather(0, 0)
        issue_scatter(0, 0)
        wait_scatter(0, 0)

        plsc.subcore_barrier()
        pltpu.sync_copy(acc_sh.at[pl.ds(row0, ROWS_PT)],
                        out_hbm.at[c, pl.ds(row0, ROWS_PT)])

    return k


_scatter_cache = {}


def _scatter(fd, src3d, dst3d):
    D = fd.shape[1]
    if D not in _scatter_cache:
        _scatter_cache[D] = _scatter_kernel(D, tc_tiling=(D % 128 == 0))
    return _scatter_cache[D](fd, src3d, dst3d)


_R = 2048


def _dinv_xd(degp, xp):
    def body(d_ref, x_ref, v_ref, xd_ref):
        i = pl.program_id(0)
        total = d_ref[0, :, 0:1] + d_ref[1, :, 0:1] + 1.0
        rowid = i * _R + lax.broadcasted_iota(jnp.int32, (_R, 1), 0)
        v = jnp.where(rowid < N, lax.rsqrt(total), 0.0)
        v_ref[...] = v
        xd_ref[...] = x_ref[...] * v

    return pl.pallas_call(
        body,
        grid=(NP // _R,),
        in_specs=[
            pl.BlockSpec((NC, _R, DEGW), lambda i: (0, i, 0)),
            pl.BlockSpec((_R, D), lambda i: (i, 0)),
        ],
        out_specs=[
            pl.BlockSpec((_R, 1), lambda i: (i, 0)),
            pl.BlockSpec((_R, D), lambda i: (i, 0)),
        ],
        out_shape=[
            jax.ShapeDtypeStruct((NP, 1), jnp.float32),
            jax.ShapeDtypeStruct((NP, D), jnp.float32),
        ],
    )(degp, xp)


def _combine(g, fd, dinv, specs):
    nw = len(specs)
    Din = fd.shape[1]

    def body(g0_ref, g1_ref, fd_ref, v_ref, *rest):
        b_refs = rest[:nw]
        w_refs = rest[nw:2 * nw]
        o_refs = rest[2 * nw:]
        v = v_ref[...]
        u = g0_ref[0] + g1_ref[0] + fd_ref[...]
        for (w_ref, b_ref, o_ref, (_, _, scale, out_w, _)) in zip(
                w_refs, b_refs, o_refs, specs):
            t = v * jnp.dot(u, w_ref[...],
                            preferred_element_type=jnp.float32) + b_ref[...]
            t = jnp.maximum(t, 0.0)
            if scale:
                t = t * v
            o_ref[...] = t[:, :out_w]

    in_specs = [
        pl.BlockSpec((1, _R, Din), lambda i: (0, i, 0)),
        pl.BlockSpec((1, _R, Din), lambda i: (1, i, 0)),
        pl.BlockSpec((_R, Din), lambda i: (i, 0)),
        pl.BlockSpec((_R, 1), lambda i: (i, 0)),
    ]
    in_specs += [pl.BlockSpec(sp[1].shape, lambda i: (0,)) for sp in specs]
    in_specs += [pl.BlockSpec(sp[0].shape, lambda i: (0, 0)) for sp in specs]
    out_specs = [pl.BlockSpec((_R, sp[3]), lambda i: (i, 0)) for sp in specs]
    out_shape = [jax.ShapeDtypeStruct((sp[4], sp[3]), jnp.float32)
                 for sp in specs]
    res = pl.pallas_call(
        body,
        grid=(NP // _R,),
        in_specs=in_specs,
        out_specs=out_specs,
        out_shape=out_shape,
    )(g, g, fd, dinv,
      *[sp[1] for sp in specs], *[sp[0] for sp in specs])
    return res


def _gram(hs):
    BI, BJ = 1024, 2560

    def body(a_ref, b_ref, o_ref):
        o_ref[...] = lax.dot_general(
            a_ref[...].astype(jnp.bfloat16), b_ref[...].astype(jnp.bfloat16),
            dimension_numbers=(((1,), (1,)), ((), ())),
            preferred_element_type=jnp.float32)

    return pl.pallas_call(
        body,
        grid=(pl.cdiv(N, BI), pl.cdiv(N, BJ)),
        in_specs=[
            pl.BlockSpec((BI, 64), lambda i, j: (i, 0)),
            pl.BlockSpec((BJ, 64), lambda i, j: (j, 0)),
        ],
        out_specs=pl.BlockSpec((BI, BJ), lambda i, j: (i, j)),
        out_shape=jax.ShapeDtypeStruct((N, N), jnp.float32),
    )(hs, hs)


def kernel(x, edge_index, W1, b1, W2, b2, W3, b3, W4, b4, W5, b5):
    pad = EPAD - E
    padidx = (N + (jnp.arange(pad, dtype=jnp.int32) % (NP - N))).astype(jnp.int32)
    src3d = jnp.concatenate([edge_index[0], padidx]).reshape(NC * NS, CPT, CHUNK)
    dst3d = jnp.concatenate([edge_index[1], padidx]).reshape(NC * NS, CPT, CHUNK)
    xp = jnp.pad(x, ((0, NP - N), (0, 0)))

    degp = _deg_partials(dst3d)
    dinv, xd = _dinv_xd(degp, xp)

    gx = _scatter(xd, src3d, dst3d)
    (hd,) = _combine(gx, xd, dinv, [(W1, b1, True, D, NP)])
    gh = _scatter(hd, src3d, dst3d)
    (zd,) = _combine(gh, hd, dinv, [(W2, b2, True, 64, NP)])
    gz = _scatter(zd, src3d, dst3d)
    a1d, hs = _combine(gz, zd, dinv, [(W3, b3, True, D, NP),
                                      (W5, b5, False, 64, N)])
    ga = _scatter(a1d, src3d, dst3d)
    (x_hat,) = _combine(ga, a1d, dinv, [(W4, b4, False, D, N)])
    a_hat = _gram(hs)
    return (x_hat, a_hat)
